# Initial kernel scaffold; baseline (speedup 1.0000x reference)
import jax
import jax.numpy as jnp
from jax.experimental import pallas as pl


def kernel(x, t, tables, W1, b1, W2, b2, W3, b3):
    raise NotImplementedError("write your pallas kernel here")


# speedup vs baseline: 29.7714x; 29.7714x over previous
import functools

import numpy as np
import jax
import jax.numpy as jnp
from jax import lax
from jax.experimental import pallas as pl
from jax.experimental.pallas import tpu as pltpu
from jax.experimental.pallas import tpu_sc as plsc

N_LEVELS = 16
F_PER_LEVEL = 2
LOG2_T = 16
TABLE_SIZE = 1 << LOG2_T
BASE_RES_ = 16
FINEST_RES_ = 512
N_DFT = 4
MLP_OUT_ = 16
N_PTS = 131072
HIDDEN_ = 64

_GROWTH = np.exp((np.log(FINEST_RES_) - np.log(BASE_RES_)) / (N_LEVELS - 1))
_RES_LIST = [float(np.floor(BASE_RES_ * _GROWTH ** l)) for l in range(N_LEVELS)]
_P1 = int(np.int32(np.uint32(2654435761)))
_P2 = int(np.int32(np.uint32(805459861)))

NC = 2
NS = 16
NW = NC * NS
P_PER_W = N_PTS // NW
CHUNK = 16
N_CHUNKS = P_PER_W // CHUNK


def _sc_embed_body(xr_hbm, yr_hbm, zr_hbm, res_hbm, tab_hbm, out_hbm,
                   xb_v, yb_v, zb_v, res_v, idx_v, w_v, rows_v, outc_v, sem):
    cid = lax.axis_index("c")
    sid = lax.axis_index("s")
    wid = sid * NC + cid
    wbase = wid * P_PER_W

    pltpu.sync_copy(xr_hbm.at[pl.ds(wbase, P_PER_W)], xb_v)
    pltpu.sync_copy(yr_hbm.at[pl.ds(wbase, P_PER_W)], yb_v)
    pltpu.sync_copy(zr_hbm.at[pl.ds(wbase, P_PER_W)], zb_v)
    pltpu.sync_copy(res_hbm, res_v)

    lanes = lax.iota(jnp.int32, 16)

    def chunk_body(ci, carry):
        base = ci * CHUNK
        px = xb_v[pl.ds(base, CHUNK)]
        py = yb_v[pl.ds(base, CHUNK)]
        pz = zb_v[pl.ds(base, CHUNK)]

        def lvl_a(l, c2):
            lsplat = jnp.full((16,), l, jnp.int32)
            r = plsc.load_gather(res_v, [lsplat])
            xs = px * r
            ys = py * r
            zs = pz * r
            xi = xs.astype(jnp.int32)
            yi = ys.astype(jnp.int32)
            zi = zs.astype(jnp.int32)
            wx1 = xs - xi.astype(jnp.float32)
            wy1 = ys - yi.astype(jnp.float32)
            wz1 = zs - zi.astype(jnp.float32)
            wx0 = 1.0 - wx1
            wy0 = 1.0 - wy1
            wz0 = 1.0 - wz1
            hx = (xi, xi + 1)
            hy = (yi * _P1, yi * _P1 + _P1)
            hz = (zi * _P2, zi * _P2 + _P2)
            wyz = (wy0 * wz0, wy0 * wz1, wy1 * wz0, wy1 * wz1)
            wx = (wx0, wx1)
            lbase = l * TABLE_SIZE
            for o in range(8):
                i, j, k = (o >> 2) & 1, (o >> 1) & 1, o & 1
                h = ((hx[i] ^ hy[j] ^ hz[k]) & 0xFFFF) + lbase
                col = lanes + (o * 16)
                plsc.store_scatter(idx_v, [lsplat, col], h)
                plsc.store_scatter(w_v, [lsplat, col], wx[i] * wyz[2 * j + k])
            return c2
        lax.fori_loop(0, N_LEVELS, lvl_a, 0, unroll=False)

        def fire(l, c2):
            pltpu.async_copy(tab_hbm.at[idx_v.at[l]], rows_v.at[l], sem)
            return c2
        lax.fori_loop(0, N_LEVELS, fire, 0, unroll=False)

        def drain(l, c2):
            pltpu.make_async_copy(tab_hbm.at[idx_v.at[l]], rows_v.at[l], sem).wait()
            return c2
        lax.fori_loop(0, N_LEVELS, drain, 0, unroll=False)

        def lvl_b(l, c2):
            lsplat = jnp.full((16,), l, jnp.int32)
            acc = [jnp.zeros((16,), jnp.float32) for _ in range(8)]
            for o in range(8):
                col = lanes + (o * 16)
                wv = plsc.load_gather(w_v, [lsplat, col])
                for f in range(8):
                    fsplat = jnp.full((16,), f, jnp.int32)
                    v = plsc.load_gather(rows_v, [lsplat, col, fsplat])
                    acc[f] = acc[f] + wv * v
            for f in range(8):
                plsc.store_scatter(outc_v, [lanes, jnp.full((16,), l * 8 + f, jnp.int32)],
                                   acc[f])
            return c2
        lax.fori_loop(0, N_LEVELS, lvl_b, 0, unroll=False)

        pltpu.sync_copy(outc_v, out_hbm.at[pl.ds(wbase + base, CHUNK), :])
        return carry

    lax.fori_loop(0, N_CHUNKS, chunk_body, 0, unroll=False)


@functools.cache
def _build_sc_embed():
    mesh = plsc.VectorSubcoreMesh(core_axis_name="c", subcore_axis_name="s")
    return pl.kernel(
        _sc_embed_body,
        out_type=jax.ShapeDtypeStruct((N_PTS, N_LEVELS * N_DFT * F_PER_LEVEL), jnp.float32),
        mesh=mesh,
        compiler_params=pltpu.CompilerParams(needs_layout_passes=False,
                                             use_tc_tiling_on_sc=False),
        scratch_types=[
            pltpu.VMEM((P_PER_W,), jnp.float32),
            pltpu.VMEM((P_PER_W,), jnp.float32),
            pltpu.VMEM((P_PER_W,), jnp.float32),
            pltpu.VMEM((N_LEVELS,), jnp.float32),
            pltpu.VMEM((N_LEVELS, 8 * CHUNK), jnp.int32),
            pltpu.VMEM((N_LEVELS, 8 * CHUNK), jnp.float32),
            pltpu.VMEM((N_LEVELS, 8 * CHUNK, N_DFT * F_PER_LEVEL), jnp.float32),
            pltpu.VMEM((CHUNK, N_LEVELS * N_DFT * F_PER_LEVEL), jnp.float32),
            pltpu.SemaphoreType.DMA,
        ],
    )


def _mlp_body(t_ref, g_ref, w1_ref, b1_ref, w2_ref, b2_ref, w3_ref, b3_ref, o_ref):
    tb = t_ref[...]
    col = lax.broadcasted_iota(jnp.int32, (1, 128), 1)
    kk = ((col % 8) // 2).astype(jnp.float32)
    basis = jnp.cos(np.float32(np.pi) * tb * kk)
    g = g_ref[...] * basis
    h = jnp.maximum(jnp.dot(g, w1_ref[...], preferred_element_type=jnp.float32)
                    + b1_ref[...], 0.0)
    h = jnp.maximum(jnp.dot(h, w2_ref[...], preferred_element_type=jnp.float32)
                    + b2_ref[...], 0.0)
    o_ref[...] = (jnp.dot(h, w3_ref[...], preferred_element_type=jnp.float32)
                  + b3_ref[...])


_BN = 1024


@functools.cache
def _build_mlp():
    d_in = N_LEVELS * N_DFT * F_PER_LEVEL
    return pl.pallas_call(
        _mlp_body,
        grid=(N_PTS // _BN,),
        in_specs=[
            pl.BlockSpec((_BN, 1), lambda i: (i, 0)),
            pl.BlockSpec((_BN, d_in), lambda i: (i, 0)),
            pl.BlockSpec((d_in, HIDDEN_), lambda i: (0, 0)),
            pl.BlockSpec((1, HIDDEN_), lambda i: (0, 0)),
            pl.BlockSpec((HIDDEN_, HIDDEN_), lambda i: (0, 0)),
            pl.BlockSpec((1, HIDDEN_), lambda i: (0, 0)),
            pl.BlockSpec((HIDDEN_, MLP_OUT_), lambda i: (0, 0)),
            pl.BlockSpec((1, MLP_OUT_), lambda i: (0, 0)),
        ],
        out_specs=pl.BlockSpec((_BN, MLP_OUT_), lambda i: (i, 0)),
        out_shape=jax.ShapeDtypeStruct((N_PTS, MLP_OUT_), jnp.float32),
    )


_COLMAP = np.array([(j // 8) * 2 + (j % 2) for j in range(128)], np.int32)


def kernel(x, t, tables, W1, b1, W2, b2, W3, b3):
    tab_flat = tables.reshape(N_LEVELS * TABLE_SIZE, N_DFT * F_PER_LEVEL)
    res = jnp.asarray(_RES_LIST, jnp.float32)
    feats = _build_sc_embed()(x[:, 0], x[:, 1], x[:, 2], res, tab_flat)
    W1e = W1[_COLMAP]
    return _build_mlp()(t[:, None], feats, W1e, b1[None], W2, b2[None], W3, b3[None])


# speedup vs baseline: 37.5281x; 1.2605x over previous
import functools

import numpy as np
import jax
import jax.numpy as jnp
from jax import lax
from jax.experimental import pallas as pl
from jax.experimental.pallas import tpu as pltpu
from jax.experimental.pallas import tpu_sc as plsc

N_LEVELS = 16
F_PER_LEVEL = 2
LOG2_T = 16
TABLE_SIZE = 1 << LOG2_T
BASE_RES_ = 16
FINEST_RES_ = 512
N_DFT = 4
MLP_OUT_ = 16
N_PTS = 131072
HIDDEN_ = 64

_GROWTH = np.exp((np.log(FINEST_RES_) - np.log(BASE_RES_)) / (N_LEVELS - 1))
_RES_LIST = [float(np.floor(BASE_RES_ * _GROWTH ** l)) for l in range(N_LEVELS)]
_P1 = int(np.int32(np.uint32(2654435761)))
_P2 = int(np.int32(np.uint32(805459861)))

NC = 2
NS = 16
NW = NC * NS
P_PER_W = N_PTS // NW
CHUNK = 16
N_CHUNKS = P_PER_W // CHUNK


def _sc_embed_body(xr_hbm, yr_hbm, zr_hbm, res_hbm, tab_hbm, out_hbm,
                   xb_v, yb_v, zb_v, res_v,
                   idx0_v, idx1_v, w0_v, w1_v, rows0_v, rows1_v, outc_v, sem):
    cid = lax.axis_index("c")
    sid = lax.axis_index("s")
    wid = sid * NC + cid
    wbase = wid * P_PER_W

    pltpu.sync_copy(xr_hbm.at[pl.ds(wbase, P_PER_W)], xb_v)
    pltpu.sync_copy(yr_hbm.at[pl.ds(wbase, P_PER_W)], yb_v)
    pltpu.sync_copy(zr_hbm.at[pl.ds(wbase, P_PER_W)], zb_v)
    pltpu.sync_copy(res_hbm, res_v)

    lanes = lax.iota(jnp.int32, 16)
    lanes8 = lanes * 8
    lanes128 = lanes * 128

    def phase_a(ci, idx_v, w_v):
        base = ci * CHUNK
        px = xb_v[pl.ds(base, CHUNK)]
        py = yb_v[pl.ds(base, CHUNK)]
        pz = zb_v[pl.ds(base, CHUNK)]

        def lvl_a(l, c2):
            lsplat = jnp.full((16,), l, jnp.int32)
            r = plsc.load_gather(res_v, [lsplat])
            xs = px * r
            ys = py * r
            zs = pz * r
            xi = xs.astype(jnp.int32)
            yi = ys.astype(jnp.int32)
            zi = zs.astype(jnp.int32)
            wx1 = xs - xi.astype(jnp.float32)
            wy1 = ys - yi.astype(jnp.float32)
            wz1 = zs - zi.astype(jnp.float32)
            wx0 = 1.0 - wx1
            wy0 = 1.0 - wy1
            wz0 = 1.0 - wz1
            hx = (xi, xi + 1)
            hy = (yi * _P1, yi * _P1 + _P1)
            hz = (zi * _P2, zi * _P2 + _P2)
            wyz = (wy0 * wz0, wy0 * wz1, wy1 * wz0, wy1 * wz1)
            wx = (wx0, wx1)
            lbase = l * TABLE_SIZE
            l128 = l * 128
            for o in range(8):
                i, j, k = (o >> 2) & 1, (o >> 1) & 1, o & 1
                h = ((hx[i] ^ hy[j] ^ hz[k]) & 0xFFFF) + lbase
                col = lanes + (o * 16)
                rowv = col + l128
                plsc.store_scatter(idx_v, [lsplat, col], h)
                plsc.store_scatter(w_v, [rowv], wx[i] * wyz[2 * j + k])
            return c2
        lax.fori_loop(0, N_LEVELS, lvl_a, 0, unroll=False)

    def fire(idx_v, rows_v):
        def f_(l, c2):
            pltpu.async_copy(tab_hbm.at[idx_v.at[l]],
                             rows_v.at[pl.ds(l * 128, 128), :], sem)
            return c2
        lax.fori_loop(0, N_LEVELS, f_, 0, unroll=False)

    def drain(idx_v, rows_v):
        def d_(l, c2):
            pltpu.make_async_copy(tab_hbm.at[idx_v.at[l]],
                                  rows_v.at[pl.ds(l * 128, 128), :], sem).wait()
            return c2
        lax.fori_loop(0, N_LEVELS, d_, 0, unroll=False)

    fsplats = [jnp.full((16,), f, jnp.int32) for f in range(8)]

    def phase_b(ci, w_v, rows_v):
        def lvl_b(l, c2):
            l128 = l * 128
            lb8 = l * 8
            acc = [jnp.zeros((16,), jnp.float32) for _ in range(8)]
            for o in range(8):
                rowv = lanes + (l128 + o * 16)
                wv = plsc.load_gather(w_v, [rowv])
                for f in range(8):
                    v = plsc.load_gather(rows_v, [rowv, fsplats[f]])
                    acc[f] = acc[f] + wv * v
            for f in range(8):
                plsc.store_scatter(outc_v, [lanes128 + (lb8 + f)], acc[f])
            return c2
        lax.fori_loop(0, N_LEVELS, lvl_b, 0, unroll=False)

        pltpu.sync_copy(outc_v,
                        out_hbm.at[pl.ds((wbase + ci * CHUNK) * 128, CHUNK * 128)])

    phase_a(0, idx0_v, w0_v)
    fire(idx0_v, rows0_v)

    def body2(j, carry):
        ci = j * 2
        phase_a(ci + 1, idx1_v, w1_v)
        fire(idx1_v, rows1_v)
        drain(idx0_v, rows0_v)
        phase_b(ci, w0_v, rows0_v)

        @pl.when(j < N_CHUNKS // 2 - 1)
        def _():
            phase_a(ci + 2, idx0_v, w0_v)
            fire(idx0_v, rows0_v)

        drain(idx1_v, rows1_v)
        phase_b(ci + 1, w1_v, rows1_v)
        return carry

    lax.fori_loop(0, N_CHUNKS // 2, body2, 0, unroll=False)


@functools.cache
def _build_sc_embed():
    mesh = plsc.VectorSubcoreMesh(core_axis_name="c", subcore_axis_name="s")
    return pl.kernel(
        _sc_embed_body,
        out_type=jax.ShapeDtypeStruct((N_PTS * N_LEVELS * N_DFT * F_PER_LEVEL,),
                                      jnp.float32),
        mesh=mesh,
        compiler_params=pltpu.CompilerParams(needs_layout_passes=False,
                                             use_tc_tiling_on_sc=False),
        scratch_types=[
            pltpu.VMEM((P_PER_W,), jnp.float32),
            pltpu.VMEM((P_PER_W,), jnp.float32),
            pltpu.VMEM((P_PER_W,), jnp.float32),
            pltpu.VMEM((N_LEVELS,), jnp.float32),
            pltpu.VMEM((N_LEVELS, 8 * CHUNK), jnp.int32),
            pltpu.VMEM((N_LEVELS, 8 * CHUNK), jnp.int32),
            pltpu.VMEM((N_LEVELS * 8 * CHUNK,), jnp.float32),
            pltpu.VMEM((N_LEVELS * 8 * CHUNK,), jnp.float32),
            pltpu.VMEM((N_LEVELS * 8 * CHUNK, N_DFT * F_PER_LEVEL), jnp.float32),
            pltpu.VMEM((N_LEVELS * 8 * CHUNK, N_DFT * F_PER_LEVEL), jnp.float32),
            pltpu.VMEM((CHUNK * N_LEVELS * N_DFT * F_PER_LEVEL,), jnp.float32),
            pltpu.SemaphoreType.DMA,
        ],
    )


def _mlp_body(t_ref, g_ref, w1_ref, b1_ref, w2_ref, b2_ref, w3_ref, b3_ref, o_ref):
    tb = t_ref[...]
    col = lax.broadcasted_iota(jnp.int32, (1, 128), 1)
    kk = ((col % 8) // 2).astype(jnp.float32)
    basis = jnp.cos(np.float32(np.pi) * tb * kk)
    g = g_ref[...] * basis
    h = jnp.maximum(jnp.dot(g, w1_ref[...], preferred_element_type=jnp.float32)
                    + b1_ref[...], 0.0)
    h = jnp.maximum(jnp.dot(h, w2_ref[...], preferred_element_type=jnp.float32)
                    + b2_ref[...], 0.0)
    o_ref[...] = (jnp.dot(h, w3_ref[...], preferred_element_type=jnp.float32)
                  + b3_ref[...])


_BN = 1024


@functools.cache
def _build_mlp():
    d_in = N_LEVELS * N_DFT * F_PER_LEVEL
    return pl.pallas_call(
        _mlp_body,
        grid=(N_PTS // _BN,),
        in_specs=[
            pl.BlockSpec((_BN, 1), lambda i: (i, 0)),
            pl.BlockSpec((_BN, d_in), lambda i: (i, 0)),
            pl.BlockSpec((d_in, HIDDEN_), lambda i: (0, 0)),
            pl.BlockSpec((1, HIDDEN_), lambda i: (0, 0)),
            pl.BlockSpec((HIDDEN_, HIDDEN_), lambda i: (0, 0)),
            pl.BlockSpec((1, HIDDEN_), lambda i: (0, 0)),
            pl.BlockSpec((HIDDEN_, MLP_OUT_), lambda i: (0, 0)),
            pl.BlockSpec((1, MLP_OUT_), lambda i: (0, 0)),
        ],
        out_specs=pl.BlockSpec((_BN, MLP_OUT_), lambda i: (i, 0)),
        out_shape=jax.ShapeDtypeStruct((N_PTS, MLP_OUT_), jnp.float32),
    )


_COLMAP = np.array([(j // 8) * 2 + (j % 2) for j in range(128)], np.int32)


def kernel(x, t, tables, W1, b1, W2, b2, W3, b3):
    tab_flat = tables.reshape(N_LEVELS * TABLE_SIZE, N_DFT * F_PER_LEVEL)
    res = jnp.asarray(_RES_LIST, jnp.float32)
    feats = _build_sc_embed()(x[:, 0], x[:, 1], x[:, 2], res, tab_flat)
    feats = feats.reshape(N_PTS, N_LEVELS * N_DFT * F_PER_LEVEL)
    W1e = W1[_COLMAP]
    return _build_mlp()(t[:, None], feats, W1e, b1[None], W2, b2[None], W3, b3[None])
